# Initial kernel scaffold; baseline (speedup 1.0000x reference)
import jax
import jax.numpy as jnp
from jax.experimental import pallas as pl


def kernel(x, base_weight, lora_a_stacked, lora_b_stacked, base_indices, embeddings_indices):
    raise NotImplementedError("write your pallas kernel here")


# speedup vs baseline: 1.8664x; 1.8664x over previous
import functools

import jax
import jax.numpy as jnp
from jax import lax
from jax.experimental import pallas as pl
from jax.experimental.pallas import tpu as pltpu
from jax.experimental.pallas import tpu_sc as plsc

_ORG_VOCAB = 100000
_EXTRA_VOCAB = 256
_EMBED_DIM = 2048
_MAX_LORAS = 8
_RANK = 16
_PACK = 128 // _RANK

_NC, _NS = 2, 16
_NW = _NC * _NS
_CHUNK = 16
_ACHUNK = 128


def _sc_gather_build(n_tok: int):
    tpw = n_tok // _NW
    nch = tpw // _CHUNK
    nach = tpw // _ACHUNK
    mesh = plsc.VectorSubcoreMesh(core_axis_name="c", subcore_axis_name="s")

    @functools.partial(
        pl.kernel,
        out_type=[
            jax.ShapeDtypeStruct((n_tok, _EMBED_DIM), jnp.float32),
            jax.ShapeDtypeStruct((n_tok, _PACK * _RANK), jnp.float32),
        ],
        mesh=mesh,
        scratch_types=[
            pltpu.VMEM((tpw,), jnp.int32),
            pltpu.VMEM((tpw,), jnp.int32),
            pltpu.VMEM((tpw, _PACK * _RANK), jnp.float32),
            pltpu.VMEM((_CHUNK, _EMBED_DIM), jnp.float32),
            pltpu.VMEM((_CHUNK, _EMBED_DIM), jnp.float32),
            pltpu.SemaphoreType.DMA,
            pltpu.SemaphoreType.DMA,
            pltpu.SemaphoreType.DMA,
        ],
    )
    def sc_gather(base_hbm, lora_a_hbm, idx_hbm, arow_hbm, rows_out, a_out,
                  idx_v, arow_v, a_v, buf0, buf1, sem0, sem1, sem_a):
        wid = lax.axis_index("s") * _NC + lax.axis_index("c")
        base = wid * tpw
        pltpu.sync_copy(idx_hbm.at[pl.ds(base, tpw)], idx_v)
        pltpu.sync_copy(arow_hbm.at[pl.ds(base, tpw)], arow_v)
        a_dmas = []
        for c in range(nach):
            a_dmas.append(pltpu.async_copy(
                lora_a_hbm.at[arow_v.at[pl.ds(c * _ACHUNK, _ACHUNK)]],
                a_v.at[pl.ds(c * _ACHUNK, _ACHUNK)], sem_a))

        bufs = (buf0, buf1)
        sems = (sem0, sem1)
        dmas = [None, None]
        dmas[0] = pltpu.async_copy(
            base_hbm.at[idx_v.at[pl.ds(0, _CHUNK)]], bufs[0], sems[0])
        for c in range(nch):
            nxt = c + 1
            if nxt < nch:
                dmas[nxt % 2] = pltpu.async_copy(
                    base_hbm.at[idx_v.at[pl.ds(nxt * _CHUNK, _CHUNK)]],
                    bufs[nxt % 2], sems[nxt % 2])
            dmas[c % 2].wait()
            pltpu.sync_copy(bufs[c % 2],
                            rows_out.at[pl.ds(base + c * _CHUNK, _CHUNK)])
        for d in a_dmas:
            d.wait()
        pltpu.sync_copy(a_v, a_out.at[pl.ds(base, tpw)])

    return sc_gather


def _tc_body(rows_ref, aw_ref, sub_ref, idx_ref, bt_ref, out_ref):
    aw = aw_ref[...]
    sub = sub_ref[...]
    idx = idx_ref[...]
    bt_blk = aw.shape[0]
    a_sel = jnp.zeros((bt_blk, _RANK), jnp.float32)
    for g in range(_PACK):
        a_sel = a_sel + jnp.where(
            sub == g, aw[:, g * _RANK:(g + 1) * _RANK], 0.0)
    cols = lax.broadcasted_iota(jnp.int32, (bt_blk, _MAX_LORAS * _RANK), 1)
    sel = (cols // _RANK) == idx
    a_exp = jnp.where(sel, jnp.concatenate([a_sel] * _MAX_LORAS, axis=1), 0.0)
    delta = jnp.dot(a_exp, bt_ref[...],
                    preferred_element_type=jnp.float32,
                    precision=lax.Precision.HIGHEST)
    out_ref[...] = rows_ref[...] + delta


def kernel(x, base_weight, lora_a_stacked, lora_b_stacked, base_indices,
           embeddings_indices):
    b, s = x.shape
    n_tok = b * s
    xf = x.reshape(n_tok).astype(jnp.int32)
    aidx = xf + embeddings_indices[1][:n_tok]
    arow = aidx >> 3
    sub2 = (aidx & (_PACK - 1)).reshape(n_tok, 1)

    lora_a_packed = lora_a_stacked.reshape(
        _MAX_LORAS * (_ORG_VOCAB + _EXTRA_VOCAB) // _PACK, _PACK * _RANK)
    bt2 = lora_b_stacked[:, 0].transpose(0, 2, 1).reshape(
        _MAX_LORAS * _RANK, _EMBED_DIM)

    rows, a_wide = _sc_gather_build(n_tok)(base_weight, lora_a_packed, xf, arow)

    bt_tok = 512
    grid = (n_tok // bt_tok,)
    idx2 = base_indices[:n_tok].reshape(n_tok, 1).astype(jnp.int32)
    out = pl.pallas_call(
        _tc_body,
        grid=grid,
        in_specs=[
            pl.BlockSpec((bt_tok, _EMBED_DIM), lambda i: (i, 0)),
            pl.BlockSpec((bt_tok, _PACK * _RANK), lambda i: (i, 0)),
            pl.BlockSpec((bt_tok, 1), lambda i: (i, 0)),
            pl.BlockSpec((bt_tok, 1), lambda i: (i, 0)),
            pl.BlockSpec((_MAX_LORAS * _RANK, _EMBED_DIM), lambda i: (0, 0)),
        ],
        out_specs=pl.BlockSpec((bt_tok, _EMBED_DIM), lambda i: (i, 0)),
        out_shape=jax.ShapeDtypeStruct((n_tok, _EMBED_DIM), jnp.float32),
    )(rows, a_wide, sub2, idx2, bt2)

    return out.reshape(b, s, _EMBED_DIM)
